# baseline (speedup 1.0000x reference)
import functools
import math

import jax
import jax.numpy as jnp
from jax import lax
from jax.experimental import pallas as pl
from jax.experimental.pallas import tpu as pltpu
from jax.experimental.pallas import tpu_sc as plsc

N = 10000
E = 320000
D = 128
H = 8
C = 64
HD = C // H
RH = 32

EP = 327680
BN = 1000
BE = 2048

_INTERPRET = False


def _head_matrix():
    r = lax.broadcasted_iota(jnp.int32, (C, H), 0)
    c = lax.broadcasted_iota(jnp.int32, (C, H), 1)
    return (r // HD == c).astype(jnp.float32)


def _node_pre_body(x_ref, wq_ref, wk_ref, wv_ref, q_ref, kv_ref):
    x = x_ref[...]
    q = jnp.dot(x, wq_ref[...], preferred_element_type=jnp.float32)
    q_ref[...] = jnp.concatenate(
        [q, jnp.zeros((q.shape[0], C), jnp.float32)], axis=1)
    k = jnp.dot(x, wk_ref[...], preferred_element_type=jnp.float32)
    v = jnp.dot(x, wv_ref[...], preferred_element_type=jnp.float32)
    kv_ref[...] = jnp.concatenate([k, v], axis=1)


def _node_pre(x, wq, wk, wv):
    return pl.pallas_call(
        _node_pre_body,
        grid=(N // BN,),
        in_specs=[
            pl.BlockSpec((BN, D), lambda i: (i, 0)),
            pl.BlockSpec((D, C), lambda i: (0, 0)),
            pl.BlockSpec((D, C), lambda i: (0, 0)),
            pl.BlockSpec((D, C), lambda i: (0, 0)),
        ],
        out_specs=[
            pl.BlockSpec((BN, D), lambda i: (i, 0)),
            pl.BlockSpec((BN, D), lambda i: (i, 0)),
        ],
        out_shape=[
            jax.ShapeDtypeStruct((N, D), jnp.float32),
            jax.ShapeDtypeStruct((N, D), jnp.float32),
        ],
        compiler_params=pltpu.CompilerParams(
            dimension_semantics=("parallel",)),
        interpret=_INTERPRET,
    )(x, wq, wk, wv)


def _edge_attn_body(geq_ref, gekv_ref, rp_ref, sc_ref,
                    rk1_ref, rk2_ref, rv1_ref, rv2_ref, out_ref):
    rp = rp_ref[...]
    dist = jnp.sqrt(jnp.sum(rp * rp, axis=1, keepdims=True))
    hk = jax.nn.relu(dist * rk1_ref[...])
    rk = jnp.dot(hk, rk2_ref[...], preferred_element_type=jnp.float32)
    hv = jax.nn.relu(dist * rv1_ref[...])
    rv = jnp.dot(hv, rv2_ref[...], preferred_element_type=jnp.float32)
    kv = gekv_ref[...]
    k = kv[:, :C] * rk
    v = kv[:, C:] * rv
    prod = geq_ref[:, :C] * k
    S = _head_matrix()
    logits = jnp.dot(prod, S, preferred_element_type=jnp.float32) / math.sqrt(HD)
    ex = jnp.exp(logits) * sc_ref[...]
    ex64 = jnp.dot(ex, S.T, preferred_element_type=jnp.float32)
    wmsg = ex64 * v
    zeros = jnp.zeros((wmsg.shape[0], 56), jnp.float32)
    out_ref[...] = jnp.concatenate([wmsg, ex, zeros], axis=1)


def _edge_attn(geq, gekv, rel_pos, scale2d, rk1, rk2, rv1, rv2):
    return pl.pallas_call(
        _edge_attn_body,
        grid=(EP // BE,),
        in_specs=[
            pl.BlockSpec((BE, D), lambda i: (i, 0)),
            pl.BlockSpec((BE, D), lambda i: (i, 0)),
            pl.BlockSpec((BE, 3), lambda i: (i, 0)),
            pl.BlockSpec((BE, 1), lambda i: (i, 0)),
            pl.BlockSpec((1, RH), lambda i: (0, 0)),
            pl.BlockSpec((RH, C), lambda i: (0, 0)),
            pl.BlockSpec((1, RH), lambda i: (0, 0)),
            pl.BlockSpec((RH, C), lambda i: (0, 0)),
        ],
        out_specs=pl.BlockSpec((BE, D), lambda i: (i, 0)),
        out_shape=jax.ShapeDtypeStruct((EP, D), jnp.float32),
        compiler_params=pltpu.CompilerParams(
            dimension_semantics=("parallel",)),
        interpret=_INTERPRET,
    )(geq, gekv, rel_pos, scale2d, rk1, rk2, rv1, rv2)


def _node_post_body(p_ref, x_ref, wo_ref, wf_ref, xn_ref, xf_ref):
    agg = p_ref[0] + p_ref[1]
    S = _head_matrix()
    den64 = jnp.dot(agg[:, C:C + H], S.T, preferred_element_type=jnp.float32)
    attn_agg = agg[:, :C] / (den64 + 1e-9)
    xn = x_ref[...] + jnp.dot(attn_agg, wo_ref[...],
                              preferred_element_type=jnp.float32)
    xn_ref[...] = xn
    xf_ref[...] = jnp.dot(xn, wf_ref[...], preferred_element_type=jnp.float32)


def _node_post(parts, x, wo, wf):
    return pl.pallas_call(
        _node_post_body,
        grid=(N // BN,),
        in_specs=[
            pl.BlockSpec((2, BN, D), lambda i: (0, i, 0)),
            pl.BlockSpec((BN, D), lambda i: (i, 0)),
            pl.BlockSpec((C, D), lambda i: (0, 0)),
            pl.BlockSpec((D, D), lambda i: (0, 0)),
        ],
        out_specs=[
            pl.BlockSpec((BN, D), lambda i: (i, 0)),
            pl.BlockSpec((BN, D), lambda i: (i, 0)),
        ],
        out_shape=[
            jax.ShapeDtypeStruct((N, D), jnp.float32),
            jax.ShapeDtypeStruct((N, D), jnp.float32),
        ],
        compiler_params=pltpu.CompilerParams(
            dimension_semantics=("parallel",)),
        interpret=_INTERPRET,
    )(parts, x, wo, wf)


def _edge_final_body(gef_ref, rp_ref, sc_ref, rf1_ref, rf2_ref, out_ref):
    rp = rp_ref[...]
    dist = jnp.sqrt(jnp.sum(rp * rp, axis=1, keepdims=True))
    hf = jax.nn.relu(dist * rf1_ref[...])
    rf = jnp.dot(hf, rf2_ref[...], preferred_element_type=jnp.float32)
    out_ref[...] = gef_ref[...] * rf * sc_ref[...]


def _edge_final(gef, rel_pos, scale2d, rf1, rf2):
    return pl.pallas_call(
        _edge_final_body,
        grid=(EP // BE,),
        in_specs=[
            pl.BlockSpec((BE, D), lambda i: (i, 0)),
            pl.BlockSpec((BE, 3), lambda i: (i, 0)),
            pl.BlockSpec((BE, 1), lambda i: (i, 0)),
            pl.BlockSpec((1, RH), lambda i: (0, 0)),
            pl.BlockSpec((RH, D), lambda i: (0, 0)),
        ],
        out_specs=pl.BlockSpec((BE, D), lambda i: (i, 0)),
        out_shape=jax.ShapeDtypeStruct((EP, D), jnp.float32),
        compiler_params=pltpu.CompilerParams(
            dimension_semantics=("parallel",)),
        interpret=_INTERPRET,
    )(gef, rel_pos, scale2d, rf1, rf2)


def _final_sum_body(p_ref, out_ref):
    out_ref[...] = p_ref[0] + p_ref[1]


def _final_sum(parts):
    return pl.pallas_call(
        _final_sum_body,
        grid=(N // BN,),
        in_specs=[pl.BlockSpec((2, BN, D), lambda i: (0, i, 0))],
        out_specs=pl.BlockSpec((BN, D), lambda i: (i, 0)),
        out_shape=jax.ShapeDtypeStruct((N, D), jnp.float32),
        compiler_params=pltpu.CompilerParams(
            dimension_semantics=("parallel",)),
        interpret=_INTERPRET,
    )(parts)


_NC = 2
_NS = 16
_NW = _NC * _NS
_EPW = EP // _NW
_CH = 128
_NCHK = _EPW // _CH
_NPT = 640


def _sc_mesh():
    return plsc.VectorSubcoreMesh(core_axis_name="c", subcore_axis_name="s")


_GATHER_CACHE = {}


def _sc_gather(streams, nbuf):
    ns = len(streams)
    flat = []
    for t, i in streams:
        flat += [t, i]
    key = (ns, nbuf)
    if key in _GATHER_CACHE:
        return _GATHER_CACHE[key](*flat)

    @functools.partial(
        pl.kernel,
        out_type=[jax.ShapeDtypeStruct((EP, D), jnp.float32)
                  for _ in range(ns)],
        mesh=_sc_mesh(),
        scratch_types=[pltpu.VMEM((_NCHK, _CH), jnp.int32)
                       for _ in range(ns)] +
                      [pltpu.VMEM((nbuf, _CH, D), jnp.float32)
                       for _ in range(ns)] +
                      [pltpu.SemaphoreType.DMA((nbuf,)),
                       pltpu.SemaphoreType.DMA((nbuf,))],
    )
    def k(*refs):
        tabs = refs[0:2 * ns:2]
        idxs = refs[1:2 * ns:2]
        outs = refs[2 * ns:3 * ns]
        idx_all = refs[3 * ns:4 * ns]
        rows = refs[4 * ns:5 * ns]
        gsems = refs[5 * ns]
        wsems = refs[5 * ns + 1]
        c = lax.axis_index("c")
        s = lax.axis_index("s")
        wid = s * _NC + c
        base = wid * _EPW

        def one_stream(i_hbm, t_hbm, o_hbm, i_v, r_v):
            pltpu.sync_copy(i_hbm.at[pl.ds(wid * _NCHK, _NCHK)], i_v)

            def fire(j, p):
                pltpu.async_copy(t_hbm.at[i_v.at[j]], r_v.at[p], gsems.at[p])

            def wait_write(p):
                pltpu.make_async_copy(r_v.at[p],
                                      o_hbm.at[pl.ds(base, _CH)],
                                      wsems.at[p]).wait()

            fire(0, 0)
            fire(1, 1)

            @pl.loop(0, _NCHK)
            def _(j):
                sj = lax.rem(j, nbuf)
                pltpu.make_async_copy(t_hbm.at[i_v.at[0]], r_v.at[sj],
                                      gsems.at[sj]).wait()
                pltpu.async_copy(r_v.at[sj],
                                 o_hbm.at[pl.ds(base + j * _CH, _CH)],
                                 wsems.at[sj])
                s2 = lax.rem(j + 2, nbuf)

                @pl.when(j >= nbuf - 2)
                def _():
                    wait_write(s2)

                @pl.when(j + 2 < _NCHK)
                def _():
                    fire(j + 2, s2)

            @pl.loop(_NCHK - nbuf + 2, _NCHK)
            def _(j):
                wait_write(lax.rem(j, nbuf))

        for j in range(ns):
            one_stream(idxs[j], tabs[j], outs[j], idx_all[j], rows[j])

    _GATHER_CACHE[key] = k
    return k(*flat)


_SCATTER_CACHE = {}


def _sc_scatter_add(rows, idx, zeros, nbuf=2):
    if nbuf in _SCATTER_CACHE:
        return _SCATTER_CACHE[nbuf](rows, idx, zeros)

    @functools.partial(
        pl.kernel,
        out_type=jax.ShapeDtypeStruct((_NC, N, D), jnp.float32),
        mesh=_sc_mesh(),
        scratch_types=[pltpu.VMEM((_NCHK, _CH), jnp.int32),
                       pltpu.VMEM((nbuf, _CH, D), jnp.float32),
                       pltpu.VMEM_SHARED((_NS * _NPT, D), jnp.float32),
                       pltpu.SemaphoreType.DMA((nbuf,)),
                       pltpu.SemaphoreType.DMA((nbuf,))],
    )
    def k(r_hbm, i_hbm, z_hbm, o_hbm, i_v, r_v, acc_sh, rsems, ssems):
        c = lax.axis_index("c")
        s = lax.axis_index("s")
        wid = s * _NC + c
        base = wid * _EPW
        pltpu.sync_copy(i_hbm.at[pl.ds(wid * _NCHK, _NCHK)], i_v)
        @pl.when(s < _NS - 1)
        def _():
            pltpu.sync_copy(z_hbm.at[pl.ds(s * _NPT, _NPT)],
                            acc_sh.at[pl.ds(s * _NPT, _NPT)])

        @pl.when(s == _NS - 1)
        def _():
            pltpu.sync_copy(z_hbm.at[pl.ds((_NS - 1) * _NPT, N - (_NS - 1) * _NPT)],
                            acc_sh.at[pl.ds((_NS - 1) * _NPT, N - (_NS - 1) * _NPT)])
        plsc.subcore_barrier()

        def load_rows(j, p):
            pltpu.async_copy(r_hbm.at[pl.ds(base + j * _CH, _CH)],
                             r_v.at[p], rsems.at[p])

        def wait_scatter(p):
            pltpu.make_async_copy(r_v.at[p], acc_sh.at[i_v.at[0]],
                                  ssems.at[p]).wait()

        load_rows(0, 0)
        load_rows(1, 1)

        @pl.loop(0, _NCHK)
        def _(j):
            sj = lax.rem(j, nbuf)
            pltpu.make_async_copy(r_hbm.at[pl.ds(base, _CH)],
                                  r_v.at[sj], rsems.at[sj]).wait()
            pltpu.async_copy(r_v.at[sj], acc_sh.at[i_v.at[j]],
                             ssems.at[sj], add=True)

            @pl.when(j + 2 < _NCHK)
            def _():
                wait_scatter(sj)
                load_rows(j + 2, sj)

        @pl.loop(_NCHK - 2, _NCHK)
        def _(j):
            wait_scatter(lax.rem(j, nbuf))

        plsc.subcore_barrier()

        @pl.when(s < _NS - 1)
        def _():
            pltpu.sync_copy(acc_sh.at[pl.ds(s * _NPT, _NPT)],
                            o_hbm.at[c, pl.ds(s * _NPT, _NPT)])

        @pl.when(s == _NS - 1)
        def _():
            pltpu.sync_copy(
                acc_sh.at[pl.ds((_NS - 1) * _NPT, N - (_NS - 1) * _NPT)],
                o_hbm.at[c, pl.ds((_NS - 1) * _NPT, N - (_NS - 1) * _NPT)])

    _SCATTER_CACHE[nbuf] = k
    return k(rows, idx, zeros)


def kernel(node_feats, edge_index, rel_pos, scale, Wq, Wk, Wv, Wo,
           Rk1, Rk2, Rv1, Rv2, Wf, Rf1, Rf2):
    pad = EP - E
    src = jnp.pad(edge_index[0].astype(jnp.int32), (0, pad)
                  ).reshape(EP // _CH, _CH)
    dst = jnp.pad(edge_index[1].astype(jnp.int32), (0, pad)
                  ).reshape(EP // _CH, _CH)
    rel_pos_p = jnp.pad(rel_pos, ((0, pad), (0, 0)))
    scale2d = jnp.pad(scale, (0, pad)).reshape(EP, 1)
    zeros = jnp.zeros((N, D), jnp.float32)

    x = node_feats
    for l in range(2):
        q, kv = _node_pre(x, Wq[l], Wk[l], Wv[l])
        geq, gekv = _sc_gather([(q, dst), (kv, src)], nbuf=3)
        packed = _edge_attn(geq, gekv, rel_pos_p, scale2d,
                            Rk1[l], Rk2[l], Rv1[l], Rv2[l])
        parts = _sc_scatter_add(packed, dst, zeros)
        if l == 0:
            x, _ = _node_post(parts, x, Wo[l], Wf)
        else:
            x, xf = _node_post(parts, x, Wo[l], Wf)

    gef, = _sc_gather([(xf, src)], nbuf=5)
    msgf = _edge_final(gef, rel_pos_p, scale2d, Rf1, Rf2)
    fparts = _sc_scatter_add(msgf, dst, zeros)
    return _final_sum(fparts)


# speedup vs baseline: 1.6512x; 1.6512x over previous
import functools
import math

import jax
import jax.numpy as jnp
from jax import lax
from jax.experimental import pallas as pl
from jax.experimental.pallas import tpu as pltpu
from jax.experimental.pallas import tpu_sc as plsc

N = 10000
E = 320000
D = 128
H = 8
C = 64
HD = C // H
RH = 32

BN = 1000
BE = 2000

_INTERPRET = False


def _head_matrix():
    r = lax.broadcasted_iota(jnp.int32, (C, H), 0)
    c = lax.broadcasted_iota(jnp.int32, (C, H), 1)
    return (r // HD == c).astype(jnp.float32)


def _node_pre_body(x_ref, wq_ref, wk_ref, wv_ref, q_ref, kv_ref):
    x = x_ref[...]
    q_ref[...] = jnp.dot(x, wq_ref[...], preferred_element_type=jnp.float32)
    k = jnp.dot(x, wk_ref[...], preferred_element_type=jnp.float32)
    v = jnp.dot(x, wv_ref[...], preferred_element_type=jnp.float32)
    kv_ref[...] = jnp.concatenate([k, v], axis=1)


def _node_pre(x, wq, wk, wv):
    return pl.pallas_call(
        _node_pre_body,
        grid=(N // BN,),
        in_specs=[
            pl.BlockSpec((BN, D), lambda i: (i, 0)),
            pl.BlockSpec((D, C), lambda i: (0, 0)),
            pl.BlockSpec((D, C), lambda i: (0, 0)),
            pl.BlockSpec((D, C), lambda i: (0, 0)),
        ],
        out_specs=[
            pl.BlockSpec((BN, C), lambda i: (i, 0)),
            pl.BlockSpec((BN, 2 * C), lambda i: (i, 0)),
        ],
        out_shape=[
            jax.ShapeDtypeStruct((N, C), jnp.float32),
            jax.ShapeDtypeStruct((N, 2 * C), jnp.float32),
        ],
        compiler_params=pltpu.CompilerParams(
            dimension_semantics=("parallel",)),
        interpret=_INTERPRET,
    )(x, wq, wk, wv)


def _edge_attn_body(geq_ref, gekv_ref, rp_ref, sc_ref,
                    rk1_ref, rk2_ref, rv1_ref, rv2_ref, out_ref):
    rp = rp_ref[...]
    dist = jnp.sqrt(jnp.sum(rp * rp, axis=1, keepdims=True))
    hk = jax.nn.relu(dist * rk1_ref[...])
    rk = jnp.dot(hk, rk2_ref[...], preferred_element_type=jnp.float32)
    hv = jax.nn.relu(dist * rv1_ref[...])
    rv = jnp.dot(hv, rv2_ref[...], preferred_element_type=jnp.float32)
    kv = gekv_ref[...]
    k = kv[:, :C] * rk
    v = kv[:, C:] * rv
    prod = geq_ref[...] * k
    S = _head_matrix()
    logits = jnp.dot(prod, S, preferred_element_type=jnp.float32) / math.sqrt(HD)
    ex = jnp.exp(logits) * sc_ref[...]
    ex64 = jnp.dot(ex, S.T, preferred_element_type=jnp.float32)
    wmsg = ex64 * v
    zeros = jnp.zeros((wmsg.shape[0], 8), jnp.float32)
    out_ref[...] = jnp.concatenate([wmsg, ex, zeros], axis=1)


def _edge_attn(geq, gekv, rel_pos, scale2d, rk1, rk2, rv1, rv2):
    ne = geq.shape[0]
    return pl.pallas_call(
        _edge_attn_body,
        grid=(ne // BE,),
        in_specs=[
            pl.BlockSpec((BE, C), lambda i: (i, 0)),
            pl.BlockSpec((BE, 2 * C), lambda i: (i, 0)),
            pl.BlockSpec((BE, 3), lambda i: (i, 0)),
            pl.BlockSpec((BE, 1), lambda i: (i, 0)),
            pl.BlockSpec((1, RH), lambda i: (0, 0)),
            pl.BlockSpec((RH, C), lambda i: (0, 0)),
            pl.BlockSpec((1, RH), lambda i: (0, 0)),
            pl.BlockSpec((RH, C), lambda i: (0, 0)),
        ],
        out_specs=pl.BlockSpec((BE, 80), lambda i: (i, 0)),
        out_shape=jax.ShapeDtypeStruct((ne, 80), jnp.float32),
        compiler_params=pltpu.CompilerParams(
            dimension_semantics=("parallel",)),
        interpret=_INTERPRET,
    )(geq, gekv, rel_pos, scale2d, rk1, rk2, rv1, rv2)


def _node_post_body(p_ref, x_ref, wo_ref, wf_ref, xn_ref, xf_ref):
    agg = p_ref[0] + p_ref[1]
    S = _head_matrix()
    den64 = jnp.dot(agg[:, C:C + H], S.T, preferred_element_type=jnp.float32)
    attn_agg = agg[:, :C] / (den64 + 1e-9)
    xn = x_ref[...] + jnp.dot(attn_agg, wo_ref[...],
                              preferred_element_type=jnp.float32)
    xn_ref[...] = xn
    xf_ref[...] = jnp.dot(xn, wf_ref[...], preferred_element_type=jnp.float32)


def _node_post(parts, x, wo, wf):
    return pl.pallas_call(
        _node_post_body,
        grid=(N // BN,),
        in_specs=[
            pl.BlockSpec((2, BN, 80), lambda i: (0, i, 0)),
            pl.BlockSpec((BN, D), lambda i: (i, 0)),
            pl.BlockSpec((C, D), lambda i: (0, 0)),
            pl.BlockSpec((D, D), lambda i: (0, 0)),
        ],
        out_specs=[
            pl.BlockSpec((BN, D), lambda i: (i, 0)),
            pl.BlockSpec((BN, D), lambda i: (i, 0)),
        ],
        out_shape=[
            jax.ShapeDtypeStruct((N, D), jnp.float32),
            jax.ShapeDtypeStruct((N, D), jnp.float32),
        ],
        compiler_params=pltpu.CompilerParams(
            dimension_semantics=("parallel",)),
        interpret=_INTERPRET,
    )(parts, x, wo, wf)


def _edge_final_body(gef_ref, rp_ref, sc_ref, rf1_ref, rf2_ref, out_ref):
    rp = rp_ref[...]
    dist = jnp.sqrt(jnp.sum(rp * rp, axis=1, keepdims=True))
    hf = jax.nn.relu(dist * rf1_ref[...])
    rf = jnp.dot(hf, rf2_ref[...], preferred_element_type=jnp.float32)
    out_ref[...] = gef_ref[...] * rf * sc_ref[...]


def _edge_final(gef, rel_pos, scale2d, rf1, rf2):
    ne = gef.shape[0]
    return pl.pallas_call(
        _edge_final_body,
        grid=(ne // BE,),
        in_specs=[
            pl.BlockSpec((BE, D), lambda i: (i, 0)),
            pl.BlockSpec((BE, 3), lambda i: (i, 0)),
            pl.BlockSpec((BE, 1), lambda i: (i, 0)),
            pl.BlockSpec((1, RH), lambda i: (0, 0)),
            pl.BlockSpec((RH, D), lambda i: (0, 0)),
        ],
        out_specs=pl.BlockSpec((BE, D), lambda i: (i, 0)),
        out_shape=jax.ShapeDtypeStruct((ne, D), jnp.float32),
        compiler_params=pltpu.CompilerParams(
            dimension_semantics=("parallel",)),
        interpret=_INTERPRET,
    )(gef, rel_pos, scale2d, rf1, rf2)


def _final_sum_body(p_ref, out_ref):
    out_ref[...] = p_ref[0] + p_ref[1]


def _final_sum(parts):
    return pl.pallas_call(
        _final_sum_body,
        grid=(N // BN,),
        in_specs=[pl.BlockSpec((2, BN, D), lambda i: (0, i, 0))],
        out_specs=pl.BlockSpec((BN, D), lambda i: (i, 0)),
        out_shape=jax.ShapeDtypeStruct((N, D), jnp.float32),
        compiler_params=pltpu.CompilerParams(
            dimension_semantics=("parallel",)),
        interpret=_INTERPRET,
    )(parts)


_NC = 2
_NS = 16
_NW = _NC * _NS
_EPW = E // _NW
_CH = 80
_NCHK = _EPW // _CH
_NBUF = 5
_NGRP = _NCHK // _NBUF
_NPT = N // _NS


def _sc_mesh():
    return plsc.VectorSubcoreMesh(core_axis_name="c", subcore_axis_name="s")


_GATHER_CACHE = {}


def _sc_gather(streams, nchk, nbuf=5):
    widths = [int(t.shape[1]) for t, _ in streams]
    dtypes = [t.dtype for t, _ in streams]
    ns = len(streams)
    ne = nchk * _NW * _CH
    flat = []
    for t, i in streams:
        flat += [t, i]
    key = (ns, tuple(widths), nchk, nbuf)
    if key in _GATHER_CACHE:
        return _GATHER_CACHE[key](*flat)

    @functools.partial(
        pl.kernel,
        out_type=[jax.ShapeDtypeStruct((ne, w), dt)
                  for w, dt in zip(widths, dtypes)],
        mesh=_sc_mesh(),
        scratch_types=[pltpu.VMEM((nchk, _CH), jnp.int32) for _ in range(ns)] +
                      [pltpu.VMEM((nbuf, _CH, w), dt)
                       for w, dt in zip(widths, dtypes)] +
                      [pltpu.SemaphoreType.DMA((nbuf,)),
                       pltpu.SemaphoreType.DMA((nbuf,))],
        compiler_params=pltpu.CompilerParams(use_tc_tiling_on_sc=False),
    )
    def k(*refs):
        tabs = refs[0:2 * ns:2]
        idxs = refs[1:2 * ns:2]
        outs = refs[2 * ns:3 * ns]
        idx_all = refs[3 * ns:4 * ns]
        rows = refs[4 * ns:5 * ns]
        gsems = refs[5 * ns]
        wsems = refs[5 * ns + 1]
        c = lax.axis_index("c")
        s = lax.axis_index("s")
        wid = s * _NC + c
        base = wid * nchk * _CH

        def one_stream(i_hbm, t_hbm, o_hbm, i_v, r_v):
            pltpu.sync_copy(i_hbm.at[pl.ds(wid * nchk, nchk)], i_v)

            def fire(j, p):
                pltpu.async_copy(t_hbm.at[i_v.at[j]], r_v.at[p], gsems.at[p])

            def wait_write(p):
                pltpu.make_async_copy(r_v.at[p],
                                      o_hbm.at[pl.ds(base, _CH)],
                                      wsems.at[p]).wait()

            fire(0, 0)
            fire(1, 1)

            @pl.loop(0, nchk)
            def _(j):
                sj = lax.rem(j, nbuf)
                pltpu.make_async_copy(t_hbm.at[i_v.at[0]], r_v.at[sj],
                                      gsems.at[sj]).wait()
                pltpu.async_copy(r_v.at[sj],
                                 o_hbm.at[pl.ds(base + j * _CH, _CH)],
                                 wsems.at[sj])
                s2 = lax.rem(j + 2, nbuf)

                @pl.when(j >= nbuf - 2)
                def _():
                    wait_write(s2)

                @pl.when(j + 2 < nchk)
                def _():
                    fire(j + 2, s2)

            @pl.loop(nchk - nbuf + 2, nchk)
            def _(j):
                wait_write(lax.rem(j, nbuf))

        for j in range(ns):
            one_stream(idxs[j], tabs[j], outs[j], idx_all[j], rows[j])

    _GATHER_CACHE[key] = k
    return k(*flat)


_SCATTER_CACHE = {}


def _sc_scatter_add(rows, idx, w, init, nchk, nbuf=5):
    key = (w, nchk, nbuf)
    if key in _SCATTER_CACHE:
        return _SCATTER_CACHE[key](rows, idx, init)

    @functools.partial(
        pl.kernel,
        out_type=jax.ShapeDtypeStruct((_NC, N, w), jnp.float32),
        mesh=_sc_mesh(),
        scratch_types=[pltpu.VMEM((nchk, _CH), jnp.int32),
                       pltpu.VMEM((nbuf, _CH, w), jnp.float32),
                       pltpu.VMEM_SHARED((N, w), jnp.float32),
                       pltpu.SemaphoreType.DMA((nbuf,)),
                       pltpu.SemaphoreType.DMA((nbuf,))],
        compiler_params=pltpu.CompilerParams(use_tc_tiling_on_sc=False),
    )
    def k(r_hbm, i_hbm, z_hbm, o_hbm, i_v, r_v, acc_sh, rsems, ssems):
        c = lax.axis_index("c")
        s = lax.axis_index("s")
        wid = s * _NC + c
        base = wid * nchk * _CH
        pltpu.sync_copy(i_hbm.at[pl.ds(wid * nchk, nchk)], i_v)
        pltpu.sync_copy(z_hbm.at[c, pl.ds(s * _NPT, _NPT)],
                        acc_sh.at[pl.ds(s * _NPT, _NPT)])
        plsc.subcore_barrier()

        def load_rows(j, p):
            pltpu.async_copy(r_hbm.at[pl.ds(base + j * _CH, _CH)],
                             r_v.at[p], rsems.at[p])

        def wait_scatter(p):
            pltpu.make_async_copy(r_v.at[p], acc_sh.at[i_v.at[0]],
                                  ssems.at[p]).wait()

        load_rows(0, 0)
        load_rows(1, 1)

        @pl.loop(0, nchk)
        def _(j):
            sj = lax.rem(j, nbuf)
            pltpu.make_async_copy(r_hbm.at[pl.ds(base, _CH)],
                                  r_v.at[sj], rsems.at[sj]).wait()
            pltpu.async_copy(r_v.at[sj], acc_sh.at[i_v.at[j]],
                             ssems.at[sj], add=True)
            s2 = lax.rem(j + 2, nbuf)

            @pl.when(j >= nbuf - 2)
            def _():
                wait_scatter(s2)

            @pl.when(j + 2 < nchk)
            def _():
                load_rows(j + 2, s2)

        @pl.loop(nchk - nbuf + 2, nchk)
        def _(j):
            wait_scatter(lax.rem(j, nbuf))
        plsc.subcore_barrier()
        pltpu.sync_copy(acc_sh.at[pl.ds(s * _NPT, _NPT)],
                        o_hbm.at[c, pl.ds(s * _NPT, _NPT)])

    _SCATTER_CACHE[key] = k
    return k(rows, idx, init)


EA = 192000
EB = E - EA
_NCHA = EA // (_NW * _CH)
_NCHB = EB // (_NW * _CH)


def kernel(node_feats, edge_index, rel_pos, scale, Wq, Wk, Wv, Wo,
           Rk1, Rk2, Rv1, Rv2, Wf, Rf1, Rf2):
    src = edge_index[0].astype(jnp.int32)
    dst = edge_index[1].astype(jnp.int32)
    srcA = src[:EA].reshape(EA // _CH, _CH)
    srcB = src[EA:].reshape(EB // _CH, _CH)
    dstA = dst[:EA].reshape(EA // _CH, _CH)
    dstB = dst[EA:].reshape(EB // _CH, _CH)
    scale2d = scale.reshape(E, 1)
    scA, scB = scale2d[:EA], scale2d[EA:]
    rpA, rpB = rel_pos[:EA], rel_pos[EA:]
    zeros80 = jnp.zeros((_NC, N, 80), jnp.float32)
    zeros128 = jnp.zeros((_NC, N, D), jnp.float32)

    x = node_feats
    for l in range(2):
        q, kv = _node_pre(x, Wq[l], Wk[l], Wv[l])
        geqA, gekvA = _sc_gather([(q, dstA), (kv, srcA)], _NCHA)
        geqB, gekvB = _sc_gather([(q, dstB), (kv, srcB)], _NCHB)
        pkA = _edge_attn(geqA, gekvA, rpA, scA,
                         Rk1[l], Rk2[l], Rv1[l], Rv2[l])
        pkB = _edge_attn(geqB, gekvB, rpB, scB,
                         Rk1[l], Rk2[l], Rv1[l], Rv2[l])
        pA = _sc_scatter_add(pkA, dstA, 80, zeros80, _NCHA)
        parts = _sc_scatter_add(pkB, dstB, 80, pA, _NCHB)
        if l == 0:
            x, _ = _node_post(parts, x, Wo[l], Wf)
        else:
            x, xf = _node_post(parts, x, Wo[l], Wf)

    gefA, = _sc_gather([(xf, srcA)], _NCHA)
    gefB, = _sc_gather([(xf, srcB)], _NCHB)
    mA = _edge_final(gefA, rpA, scA, Rf1, Rf2)
    mB = _edge_final(gefB, rpB, scB, Rf1, Rf2)
    fA = _sc_scatter_add(mA, dstA, D, zeros128, _NCHA, nbuf=3)
    fparts = _sc_scatter_add(mB, dstB, D, fA, _NCHB, nbuf=3)
    return _final_sum(fparts)
